# Initial kernel scaffold; baseline (speedup 1.0000x reference)
import jax
import jax.numpy as jnp
from jax.experimental import pallas as pl


def kernel(x, edge_index, W, b):
    raise NotImplementedError("write your pallas kernel here")


# speedup vs baseline: 21.9723x; 21.9723x over previous
import functools

import jax
import jax.numpy as jnp
from jax import lax
from jax.experimental import pallas as pl
from jax.experimental.pallas import tpu as pltpu
from jax.experimental.pallas import tpu_sc as plsc

N = 10000
E = 320000
D = 128
NPAD = 10240
NB = NPAD // 128
CH = 128
TCH = E // CH
NC = 2
NS = 16
NW = NC * NS
PER_TILE_N = NPAD // NS
ZR = 64

_mesh = functools.partial(
    plsc.VectorSubcoreMesh, core_axis_name="c", subcore_axis_name="s")


def _deg_body(dst_hbm, ones_hbm, zeros_hbm, out_hbm, idxv, onesv, stagev,
              deg_sp, sem):
    c = lax.axis_index("c")
    s = lax.axis_index("s")
    wid = c * NS + s
    pltpu.sync_copy(ones_hbm, onesv)
    pltpu.sync_copy(zeros_hbm, stagev)
    pltpu.sync_copy(stagev, deg_sp.at[pl.ds(s * PER_TILE_N, PER_TILE_N)])
    plsc.subcore_barrier()
    lo = (wid * TCH) // NW
    hi = ((wid + 1) * TCH) // NW

    def body(g, carry):
        pltpu.sync_copy(dst_hbm.at[pl.ds(g * CH, CH)], idxv)
        pltpu.sync_copy(onesv, deg_sp.at[idxv], add=True)
        return carry

    lax.fori_loop(lo, hi, body, 0)
    plsc.subcore_barrier()
    pltpu.sync_copy(deg_sp.at[pl.ds(s * PER_TILE_N, PER_TILE_N)], stagev)
    pltpu.sync_copy(stagev, out_hbm.at[c, pl.ds(s * PER_TILE_N, PER_TILE_N)])


def _deg_call(dst, ones1, zeros1):
    return pl.kernel(
        _deg_body,
        out_type=jax.ShapeDtypeStruct((NC, NPAD), jnp.float32),
        mesh=_mesh(),
        scratch_types=[
            pltpu.VMEM((CH,), jnp.int32),
            pltpu.VMEM((CH,), jnp.float32),
            pltpu.VMEM((PER_TILE_N,), jnp.float32),
            pltpu.VMEM_SHARED((NPAD,), jnp.float32),
            pltpu.SemaphoreType.DMA,
        ],
    )(dst, ones1, zeros1)


def _dot(a, b):
    return lax.dot_general(a, b, (((1,), (0,)), ((), ())),
                           preferred_element_type=jnp.float32)


def _mm_body(x_ref, w_ref, dp_ref, eye_ref, xw_ref, y_ref):
    deg = dp_ref[0] + dp_ref[1] + 1.0
    dis = lax.rsqrt(deg)
    xw = lax.dot_general(x_ref[...], w_ref[...], (((1,), (1,)), ((), ())),
                         preferred_element_type=jnp.float32)
    xw_ref[...] = xw
    eye = eye_ref[...]
    for j in range(8):
        diagm = dis[j:j + 1, :] * eye
        y_ref[128 * j:128 * (j + 1), :] = _dot(diagm, xw[128 * j:128 * (j + 1), :])


def _mm_call(x_pad, w, dp3, eye):
    return pl.pallas_call(
        _mm_body,
        grid=(10,),
        in_specs=[
            pl.BlockSpec((1024, D), lambda i: (i, 0)),
            pl.BlockSpec((D, D), lambda i: (0, 0)),
            pl.BlockSpec((NC, 8, 128), lambda i: (0, i, 0)),
            pl.BlockSpec((D, D), lambda i: (0, 0)),
        ],
        out_specs=[
            pl.BlockSpec((1024, D), lambda i: (i, 0)),
            pl.BlockSpec((1024, D), lambda i: (i, 0)),
        ],
        out_shape=[
            jax.ShapeDtypeStruct((NPAD, D), jnp.float32),
            jax.ShapeDtypeStruct((NPAD, D), jnp.float32),
        ],
    )(x_pad, w, dp3, eye)


def _edge_body(y_hbm, src_hbm, dst_hbm, zeros_hbm, out_hbm, srcv, dstv, rows,
               stage, acc_sp, sem):
    c = lax.axis_index("c")
    s = lax.axis_index("s")
    wid = c * NS + s
    pltpu.sync_copy(zeros_hbm, stage)
    for k in range(PER_TILE_N // ZR):
        pltpu.sync_copy(stage, acc_sp.at[pl.ds(s * PER_TILE_N + k * ZR, ZR)])
    plsc.subcore_barrier()
    lo = (wid * TCH) // NW
    hi = ((wid + 1) * TCH) // NW

    def body(g, carry):
        base = g * CH
        pltpu.sync_copy(src_hbm.at[pl.ds(base, CH)], srcv)
        pltpu.sync_copy(dst_hbm.at[pl.ds(base, CH)], dstv)
        pltpu.async_copy(y_hbm.at[srcv], rows, sem).wait()
        pltpu.sync_copy(rows, acc_sp.at[dstv], add=True)
        return carry

    lax.fori_loop(lo, hi, body, 0)
    plsc.subcore_barrier()
    for k in range(PER_TILE_N // ZR):
        pltpu.sync_copy(acc_sp.at[pl.ds(s * PER_TILE_N + k * ZR, ZR)], stage)
        pltpu.sync_copy(stage, out_hbm.at[c, pl.ds(s * PER_TILE_N + k * ZR, ZR)])


def _edge_call(y, src, dst, zeros2):
    return pl.kernel(
        _edge_body,
        out_type=jax.ShapeDtypeStruct((NC, NPAD, D), jnp.float32),
        mesh=_mesh(),
        scratch_types=[
            pltpu.VMEM((CH,), jnp.int32),
            pltpu.VMEM((CH,), jnp.int32),
            pltpu.VMEM((CH, D), jnp.float32),
            pltpu.VMEM((ZR, D), jnp.float32),
            pltpu.VMEM_SHARED((NPAD, D), jnp.float32),
            pltpu.SemaphoreType.DMA,
        ],
    )(y, src, dst, zeros2)


def _ep_body(acc_ref, xw_ref, dp_ref, b_ref, eye_ref, out_ref):
    deg = dp_ref[0] + dp_ref[1] + 1.0
    dis = lax.rsqrt(deg)
    invd = 1.0 / deg
    eye = eye_ref[...]
    acc = acc_ref[0] + acc_ref[1]
    for j in range(8):
        sl = slice(128 * j, 128 * (j + 1))
        dism = dis[j:j + 1, :] * eye
        invm = invd[j:j + 1, :] * eye
        h = _dot(dism, acc[sl, :]) + _dot(invm, xw_ref[sl, :]) + b_ref[0:1, :]
        out_ref[sl, :] = jnp.maximum(h, 0.0)


def _ep_call(accs, xw, dp3, b8, eye):
    return pl.pallas_call(
        _ep_body,
        grid=(10,),
        in_specs=[
            pl.BlockSpec((NC, 1024, D), lambda i: (0, i, 0)),
            pl.BlockSpec((1024, D), lambda i: (i, 0)),
            pl.BlockSpec((NC, 8, 128), lambda i: (0, i, 0)),
            pl.BlockSpec((8, D), lambda i: (0, 0)),
            pl.BlockSpec((D, D), lambda i: (0, 0)),
        ],
        out_specs=pl.BlockSpec((1024, D), lambda i: (i, 0)),
        out_shape=jax.ShapeDtypeStruct((NPAD, D), jnp.float32),
    )(accs, xw, dp3, b8, eye)


def kernel(x, edge_index, W, b):
    src = edge_index[0]
    dst = edge_index[1]
    x_pad = jnp.pad(x, ((0, NPAD - N), (0, 0)))
    ones1 = jnp.ones((CH,), jnp.float32)
    zeros1 = jnp.zeros((PER_TILE_N,), jnp.float32)
    zeros2 = jnp.zeros((ZR, D), jnp.float32)
    b8 = jnp.broadcast_to(b[None, :], (8, D))
    eye = jnp.eye(D, dtype=jnp.float32)

    dpart = _deg_call(dst, ones1, zeros1)
    dp3 = dpart.reshape(NC, NB, 128)
    xw, y = _mm_call(x_pad, W, dp3, eye)
    accs = _edge_call(y, src, dst, zeros2)
    out = _ep_call(accs, xw, dp3, b8, eye)
    return out[:N]
